# baseline (speedup 1.0000x reference)
import functools

import jax
import jax.numpy as jnp
from jax import lax
from jax.experimental import pallas as pl
from jax.experimental.pallas import tpu as pltpu
from jax.experimental.pallas import tpu_sc as plsc

N_NODES = 10000
N_EDGES = 320000
F = 128
NC, NS = 2, 16
CHUNK = 80
EDGES_PER_TILE = N_EDGES // (NC * NS)
NCHUNK = EDGES_PER_TILE // CHUNK
ROWS_A = 624
TAIL = N_NODES - NS * ROWS_A
ZROWS = 208
DEG_W = 16

def _fill_f32(ref, rows, width, value):
    per_row = width // 16

    def body(i, _):
        r = i // per_row
        col = (i % per_row) * 16
        ref[r, pl.ds(col, 16)] = jnp.full((16,), value, jnp.float32)
        return 0

    lax.fori_loop(0, rows * per_row, body, 0)


def _deg_body(dst_hbm, deg_hbm, dst_v, ones_v, zero_v, dsem, deg_sh):
    c = lax.axis_index("c")
    s = lax.axis_index("s")
    _fill_f32(ones_v, CHUNK, DEG_W, 1.0)
    _fill_f32(zero_v, ZROWS, DEG_W, 0.0)
    base = s * ROWS_A
    for k in range(ROWS_A // ZROWS):
        pltpu.sync_copy(zero_v, deg_sh.at[pl.ds(base + k * ZROWS, ZROWS)])

    @pl.when(s == NS - 1)
    def _():
        pltpu.sync_copy(zero_v.at[pl.ds(0, TAIL)],
                        deg_sh.at[pl.ds(NS * ROWS_A, TAIL)])

    pltpu.sync_copy(dst_hbm.at[c, s], dst_v)
    plsc.subcore_barrier()

    GRP = 5
    assert NCHUNK % GRP == 0

    def group(g, _):
        for i in range(GRP):
            pltpu.async_copy(ones_v, deg_sh.at[dst_v.at[g * GRP + i]], dsem,
                             add=True)
        for i in range(GRP):
            pltpu.make_async_copy(ones_v, deg_sh.at[dst_v.at[g * GRP + i]],
                                  dsem).wait()
        return 0

    lax.fori_loop(0, NCHUNK // GRP, group, 0)
    plsc.subcore_barrier()
    pltpu.sync_copy(
        deg_sh.at[pl.ds(base, ROWS_A)],
        deg_hbm.at[c, pl.ds(base, ROWS_A)],
    )

    @pl.when(s == NS - 1)
    def _():
        pltpu.sync_copy(
            deg_sh.at[pl.ds(NS * ROWS_A, TAIL)],
            deg_hbm.at[c, pl.ds(NS * ROWS_A, TAIL)],
        )


NBUF = 3


def _spmv_body(y_hbm, src_hbm, dst_hbm, out_hbm, src_v, dst_v,
               r0, r1, r2, g0, g1, g2, t0, t1, t2, z_sh):
    c = lax.axis_index("c")
    s = lax.axis_index("s")
    rows = [r0, r1, r2]
    gsem = [g0, g1, g2]
    ssem = [t0, t1, t2]
    _fill_f32(r0, CHUNK, F, 0.0)
    base = s * ROWS_A
    for k in range(ROWS_A // CHUNK):
        pltpu.sync_copy(r0, z_sh.at[pl.ds(base + k * CHUNK, CHUNK)])
    rem = ROWS_A - (ROWS_A // CHUNK) * CHUNK
    pltpu.sync_copy(r0.at[pl.ds(0, rem)],
                    z_sh.at[pl.ds(base + ROWS_A - rem, rem)])

    @pl.when(s == NS - 1)
    def _():
        pltpu.sync_copy(r0.at[pl.ds(0, TAIL)],
                        z_sh.at[pl.ds(NS * ROWS_A, TAIL)])

    pltpu.sync_copy(src_hbm.at[c, s], src_v)
    pltpu.sync_copy(dst_hbm.at[c, s], dst_v)
    plsc.subcore_barrier()

    for b in range(NBUF - 1):
        pltpu.async_copy(y_hbm.at[src_v.at[b]], rows[b], gsem[b])

    def ring(g, _):
        for b in range(NBUF):
            j = g * NBUF + b
            pltpu.make_async_copy(y_hbm.at[src_v.at[j]], rows[b], gsem[b]).wait()
            pltpu.async_copy(rows[b], z_sh.at[dst_v.at[j]], ssem[b], add=True)
            nxt = j + NBUF - 1
            nb = (b + NBUF - 1) % NBUF

            @pl.when((nxt < NCHUNK) & (j >= 1))
            def _():
                pltpu.make_async_copy(rows[nb], z_sh.at[dst_v.at[j - 1]],
                                      ssem[nb]).wait()

            @pl.when(nxt < NCHUNK)
            def _():
                pltpu.async_copy(y_hbm.at[src_v.at[nxt]], rows[nb], gsem[nb])

        return 0

    lax.fori_loop(0, NCHUNK // NBUF, ring, 0)
    for j in range((NCHUNK // NBUF) * NBUF, NCHUNK):
        b = j % NBUF
        pltpu.make_async_copy(y_hbm.at[src_v.at[j]], rows[b], gsem[b]).wait()
        pltpu.async_copy(rows[b], z_sh.at[dst_v.at[j]], ssem[b], add=True)
    last = {}
    for j in range(NCHUNK):
        last[j % NBUF] = j
    for b, j in last.items():
        pltpu.make_async_copy(rows[b], z_sh.at[dst_v.at[j]], ssem[b]).wait()
    plsc.subcore_barrier()
    pltpu.sync_copy(
        z_sh.at[pl.ds(base, ROWS_A)],
        out_hbm.at[c, pl.ds(base, ROWS_A)],
    )

    @pl.when(s == NS - 1)
    def _():
        pltpu.sync_copy(
            z_sh.at[pl.ds(NS * ROWS_A, TAIL)],
            out_hbm.at[c, pl.ds(NS * ROWS_A, TAIL)],
        )


BLK = 1000


def _dinv_of(p0d, p1d):
    d = p0d[:, 0:1] + p1d[:, 0:1]
    return jnp.where(d > 0, lax.rsqrt(jnp.maximum(d, 1.0)), 0.0)


def _prep_body(p0d_ref, p1d_ref, x_ref, y_ref):
    dinv = _dinv_of(p0d_ref[...], p1d_ref[...])
    y_ref[...] = x_ref[...] * dinv


def _layer_body(h_ref, z0_ref, z1_ref, p0d_ref, p1d_ref, w0_ref, w1_ref, b_ref,
                hn_ref, y_ref):
    dinv = _dinv_of(p0d_ref[...], p1d_ref[...])
    ahat = (z0_ref[...] + z1_ref[...]) * dinv
    out = (
        jnp.dot(h_ref[...], w0_ref[...], preferred_element_type=jnp.float32)
        - jnp.dot(ahat, w1_ref[...], preferred_element_type=jnp.float32)
        + b_ref[...]
    )
    hn = jnp.maximum(out, 0.0)
    hn_ref[...] = hn
    y_ref[...] = hn * dinv


def _final_body(h_ref, z0_ref, z1_ref, p0d_ref, p1d_ref, w0_ref, w1_ref, b_ref,
                out_ref):
    dinv = _dinv_of(p0d_ref[...], p1d_ref[...])
    ahat = (z0_ref[...] + z1_ref[...]) * dinv
    out_ref[...] = (
        jnp.dot(h_ref[...], w0_ref[...], preferred_element_type=jnp.float32)
        - jnp.dot(ahat, w1_ref[...], preferred_element_type=jnp.float32)
        + b_ref[...]
    )


_row_blk = lambda w: pl.BlockSpec((BLK, w), lambda i: (i, 0))
_full_w = pl.BlockSpec((F, F), lambda i: (0, 0))
_full_b = pl.BlockSpec((1, F), lambda i: (0, 0))
_GRID = (N_NODES // BLK,)
_f32 = jnp.float32

_prep_call = pl.pallas_call(
    _prep_body,
    grid=_GRID,
    in_specs=[_row_blk(DEG_W), _row_blk(DEG_W), _row_blk(F)],
    out_specs=_row_blk(F),
    out_shape=jax.ShapeDtypeStruct((N_NODES, F), _f32),
)

_layer_call = pl.pallas_call(
    _layer_body,
    grid=_GRID,
    in_specs=[_row_blk(F), _row_blk(F), _row_blk(F), _row_blk(DEG_W),
              _row_blk(DEG_W), _full_w, _full_w, _full_b],
    out_specs=[_row_blk(F), _row_blk(F)],
    out_shape=[jax.ShapeDtypeStruct((N_NODES, F), _f32),
               jax.ShapeDtypeStruct((N_NODES, F), _f32)],
)

_final_call = pl.pallas_call(
    _final_body,
    grid=_GRID,
    in_specs=[_row_blk(F), _row_blk(F), _row_blk(F), _row_blk(DEG_W),
              _row_blk(DEG_W), _full_w, _full_w, _full_b],
    out_specs=_row_blk(F),
    out_shape=jax.ShapeDtypeStruct((N_NODES, F), _f32),
)


@functools.lru_cache(maxsize=1)
def _sc_kernels():
    mesh = plsc.VectorSubcoreMesh(
        core_axis_name="c", subcore_axis_name="s", num_cores=NC, num_subcores=NS
    )
    params = pltpu.CompilerParams(use_tc_tiling_on_sc=False)
    deg_kernel = pl.kernel(
        _deg_body,
        compiler_params=params,
        out_type=jax.ShapeDtypeStruct((NC, N_NODES, DEG_W), jnp.float32),
        mesh=mesh,
        scratch_types=[
            pltpu.VMEM((NCHUNK, CHUNK), jnp.int32),
            pltpu.VMEM((CHUNK, DEG_W), jnp.float32),
            pltpu.VMEM((ZROWS, DEG_W), jnp.float32),
            pltpu.SemaphoreType.DMA,
            pltpu.VMEM_SHARED((N_NODES, DEG_W), jnp.float32),
        ],
    )
    spmv_kernel = pl.kernel(
        _spmv_body,
        compiler_params=params,
        out_type=jax.ShapeDtypeStruct((NC, N_NODES, F), jnp.float32),
        mesh=mesh,
        scratch_types=(
            [pltpu.VMEM((NCHUNK, CHUNK), jnp.int32)] * 2
            + [pltpu.VMEM((CHUNK, F), jnp.float32)] * NBUF
            + [pltpu.SemaphoreType.DMA] * (2 * NBUF)
            + [pltpu.VMEM_SHARED((N_NODES, F), jnp.float32)]
        ),
    )
    return deg_kernel, spmv_kernel


def kernel(x, edge_index, W1, b1, W2, b2):
    deg_k, spmv_k = _sc_kernels()
    src = edge_index[0].astype(jnp.int32).reshape(NC, NS, NCHUNK, CHUNK)
    dst = edge_index[1].astype(jnp.int32).reshape(NC, NS, NCHUNK, CHUNK)

    degp = deg_k(dst)
    p0d, p1d = degp[0], degp[1]

    y = _prep_call(p0d, p1d, x)
    h = x
    b1r = b1.reshape(1, F)
    for _ in range(2):
        zp = spmv_k(y, src, dst)
        h, y = _layer_call(h, zp[0], zp[1], p0d, p1d, W1[0], W1[1], b1r)
    zp = spmv_k(y, src, dst)
    return _final_call(h, zp[0], zp[1], p0d, p1d, W2[0], W2[1], b2.reshape(1, F))


# speedup vs baseline: 1.0514x; 1.0514x over previous
import functools

import jax
import jax.numpy as jnp
from jax import lax
from jax.experimental import pallas as pl
from jax.experimental.pallas import tpu as pltpu
from jax.experimental.pallas import tpu_sc as plsc

N_NODES = 10000
N_EDGES = 320000
F = 128
NC, NS = 2, 16
CHUNK = 40
EDGES_PER_TILE = N_EDGES // (NC * NS)
NCHUNK = EDGES_PER_TILE // CHUNK
ROWS_A = 624
TAIL = N_NODES - NS * ROWS_A
ZROWS = 208
DEG_W = 16

def _fill_f32(ref, rows, width, value):
    per_row = width // 16

    def body(i, _):
        r = i // per_row
        col = (i % per_row) * 16
        ref[r, pl.ds(col, 16)] = jnp.full((16,), value, jnp.float32)
        return 0

    lax.fori_loop(0, rows * per_row, body, 0)


def _deg_body(dst_hbm, deg_hbm, dst_v, ones_v, zero_v, dsem, deg_sh):
    c = lax.axis_index("c")
    s = lax.axis_index("s")
    _fill_f32(ones_v, CHUNK, DEG_W, 1.0)
    _fill_f32(zero_v, ZROWS, DEG_W, 0.0)
    base = s * ROWS_A
    for k in range(ROWS_A // ZROWS):
        pltpu.sync_copy(zero_v, deg_sh.at[pl.ds(base + k * ZROWS, ZROWS)])

    @pl.when(s == NS - 1)
    def _():
        pltpu.sync_copy(zero_v.at[pl.ds(0, TAIL)],
                        deg_sh.at[pl.ds(NS * ROWS_A, TAIL)])

    pltpu.sync_copy(dst_hbm.at[c, s], dst_v)
    plsc.subcore_barrier()

    GRP = 5
    assert NCHUNK % GRP == 0

    def group(g, _):
        for i in range(GRP):
            pltpu.async_copy(ones_v, deg_sh.at[dst_v.at[g * GRP + i]], dsem,
                             add=True)
        for i in range(GRP):
            pltpu.make_async_copy(ones_v, deg_sh.at[dst_v.at[g * GRP + i]],
                                  dsem).wait()
        return 0

    lax.fori_loop(0, NCHUNK // GRP, group, 0)
    plsc.subcore_barrier()
    pltpu.sync_copy(
        deg_sh.at[pl.ds(base, ROWS_A)],
        deg_hbm.at[c, pl.ds(base, ROWS_A)],
    )

    @pl.when(s == NS - 1)
    def _():
        pltpu.sync_copy(
            deg_sh.at[pl.ds(NS * ROWS_A, TAIL)],
            deg_hbm.at[c, pl.ds(NS * ROWS_A, TAIL)],
        )


NBUF = 6


def _spmv_body(y_hbm, src_hbm, dst_hbm, out_hbm, src_v, dst_v,
               r0, r1, r2, r3, r4, r5, g0, g1, g2, g3, g4, g5,
               t0, t1, t2, t3, t4, t5, z_sh):
    c = lax.axis_index("c")
    s = lax.axis_index("s")
    rows = [r0, r1, r2, r3, r4, r5]
    gsem = [g0, g1, g2, g3, g4, g5]
    ssem = [t0, t1, t2, t3, t4, t5]
    _fill_f32(r0, CHUNK, F, 0.0)
    base = s * ROWS_A
    for k in range(ROWS_A // CHUNK):
        pltpu.sync_copy(r0, z_sh.at[pl.ds(base + k * CHUNK, CHUNK)])
    rem = ROWS_A - (ROWS_A // CHUNK) * CHUNK
    pltpu.sync_copy(r0.at[pl.ds(0, rem)],
                    z_sh.at[pl.ds(base + ROWS_A - rem, rem)])

    @pl.when(s == NS - 1)
    def _():
        pltpu.sync_copy(r0.at[pl.ds(0, TAIL)],
                        z_sh.at[pl.ds(NS * ROWS_A, TAIL)])

    pltpu.sync_copy(src_hbm.at[c, s], src_v)
    pltpu.sync_copy(dst_hbm.at[c, s], dst_v)
    plsc.subcore_barrier()

    for b in range(NBUF - 1):
        pltpu.async_copy(y_hbm.at[src_v.at[b]], rows[b], gsem[b])

    def ring(g, _):
        for b in range(NBUF):
            j = g * NBUF + b
            pltpu.make_async_copy(y_hbm.at[src_v.at[j]], rows[b], gsem[b]).wait()
            pltpu.async_copy(rows[b], z_sh.at[dst_v.at[j]], ssem[b], add=True)
            nxt = j + NBUF - 1
            nb = (b + NBUF - 1) % NBUF

            @pl.when((nxt < NCHUNK) & (j >= 1))
            def _():
                pltpu.make_async_copy(rows[nb], z_sh.at[dst_v.at[j - 1]],
                                      ssem[nb]).wait()

            @pl.when(nxt < NCHUNK)
            def _():
                pltpu.async_copy(y_hbm.at[src_v.at[nxt]], rows[nb], gsem[nb])

        return 0

    lax.fori_loop(0, NCHUNK // NBUF, ring, 0)
    for j in range((NCHUNK // NBUF) * NBUF, NCHUNK):
        b = j % NBUF
        pltpu.make_async_copy(y_hbm.at[src_v.at[j]], rows[b], gsem[b]).wait()
        pltpu.async_copy(rows[b], z_sh.at[dst_v.at[j]], ssem[b], add=True)
    last = {}
    for j in range(NCHUNK):
        last[j % NBUF] = j
    for b, j in last.items():
        pltpu.make_async_copy(rows[b], z_sh.at[dst_v.at[j]], ssem[b]).wait()
    plsc.subcore_barrier()
    pltpu.sync_copy(
        z_sh.at[pl.ds(base, ROWS_A)],
        out_hbm.at[c, pl.ds(base, ROWS_A)],
    )

    @pl.when(s == NS - 1)
    def _():
        pltpu.sync_copy(
            z_sh.at[pl.ds(NS * ROWS_A, TAIL)],
            out_hbm.at[c, pl.ds(NS * ROWS_A, TAIL)],
        )


BLK = 1000


def _dinv_of(p0d, p1d):
    d = p0d[:, 0:1] + p1d[:, 0:1]
    return jnp.where(d > 0, lax.rsqrt(jnp.maximum(d, 1.0)), 0.0)


def _prep_body(p0d_ref, p1d_ref, x_ref, y_ref):
    dinv = _dinv_of(p0d_ref[...], p1d_ref[...])
    y_ref[...] = x_ref[...] * dinv


def _layer_body(h_ref, z0_ref, z1_ref, p0d_ref, p1d_ref, w0_ref, w1_ref, b_ref,
                hn_ref, y_ref):
    dinv = _dinv_of(p0d_ref[...], p1d_ref[...])
    ahat = (z0_ref[...] + z1_ref[...]) * dinv
    out = (
        jnp.dot(h_ref[...], w0_ref[...], preferred_element_type=jnp.float32)
        - jnp.dot(ahat, w1_ref[...], preferred_element_type=jnp.float32)
        + b_ref[...]
    )
    hn = jnp.maximum(out, 0.0)
    hn_ref[...] = hn
    y_ref[...] = hn * dinv


def _final_body(h_ref, z0_ref, z1_ref, p0d_ref, p1d_ref, w0_ref, w1_ref, b_ref,
                out_ref):
    dinv = _dinv_of(p0d_ref[...], p1d_ref[...])
    ahat = (z0_ref[...] + z1_ref[...]) * dinv
    out_ref[...] = (
        jnp.dot(h_ref[...], w0_ref[...], preferred_element_type=jnp.float32)
        - jnp.dot(ahat, w1_ref[...], preferred_element_type=jnp.float32)
        + b_ref[...]
    )


_row_blk = lambda w: pl.BlockSpec((BLK, w), lambda i: (i, 0))
_full_w = pl.BlockSpec((F, F), lambda i: (0, 0))
_full_b = pl.BlockSpec((1, F), lambda i: (0, 0))
_GRID = (N_NODES // BLK,)
_f32 = jnp.float32

_prep_call = pl.pallas_call(
    _prep_body,
    grid=_GRID,
    in_specs=[_row_blk(DEG_W), _row_blk(DEG_W), _row_blk(F)],
    out_specs=_row_blk(F),
    out_shape=jax.ShapeDtypeStruct((N_NODES, F), _f32),
)

_layer_call = pl.pallas_call(
    _layer_body,
    grid=_GRID,
    in_specs=[_row_blk(F), _row_blk(F), _row_blk(F), _row_blk(DEG_W),
              _row_blk(DEG_W), _full_w, _full_w, _full_b],
    out_specs=[_row_blk(F), _row_blk(F)],
    out_shape=[jax.ShapeDtypeStruct((N_NODES, F), _f32),
               jax.ShapeDtypeStruct((N_NODES, F), _f32)],
)

_final_call = pl.pallas_call(
    _final_body,
    grid=_GRID,
    in_specs=[_row_blk(F), _row_blk(F), _row_blk(F), _row_blk(DEG_W),
              _row_blk(DEG_W), _full_w, _full_w, _full_b],
    out_specs=_row_blk(F),
    out_shape=jax.ShapeDtypeStruct((N_NODES, F), _f32),
)


@functools.lru_cache(maxsize=1)
def _sc_kernels():
    mesh = plsc.VectorSubcoreMesh(
        core_axis_name="c", subcore_axis_name="s", num_cores=NC, num_subcores=NS
    )
    params = pltpu.CompilerParams(use_tc_tiling_on_sc=False)
    deg_kernel = pl.kernel(
        _deg_body,
        compiler_params=params,
        out_type=jax.ShapeDtypeStruct((NC, N_NODES, DEG_W), jnp.float32),
        mesh=mesh,
        scratch_types=[
            pltpu.VMEM((NCHUNK, CHUNK), jnp.int32),
            pltpu.VMEM((CHUNK, DEG_W), jnp.float32),
            pltpu.VMEM((ZROWS, DEG_W), jnp.float32),
            pltpu.SemaphoreType.DMA,
            pltpu.VMEM_SHARED((N_NODES, DEG_W), jnp.float32),
        ],
    )
    spmv_kernel = pl.kernel(
        _spmv_body,
        compiler_params=params,
        out_type=jax.ShapeDtypeStruct((NC, N_NODES, F), jnp.float32),
        mesh=mesh,
        scratch_types=(
            [pltpu.VMEM((NCHUNK, CHUNK), jnp.int32)] * 2
            + [pltpu.VMEM((CHUNK, F), jnp.float32)] * NBUF
            + [pltpu.SemaphoreType.DMA] * (2 * NBUF)
            + [pltpu.VMEM_SHARED((N_NODES, F), jnp.float32)]
        ),
    )
    return deg_kernel, spmv_kernel


def kernel(x, edge_index, W1, b1, W2, b2):
    deg_k, spmv_k = _sc_kernels()
    src = edge_index[0].astype(jnp.int32).reshape(NC, NS, NCHUNK, CHUNK)
    dst = edge_index[1].astype(jnp.int32).reshape(NC, NS, NCHUNK, CHUNK)

    degp = deg_k(dst)
    p0d, p1d = degp[0], degp[1]

    y = _prep_call(p0d, p1d, x)
    h = x
    b1r = b1.reshape(1, F)
    for _ in range(2):
        zp = spmv_k(y, src, dst)
        h, y = _layer_call(h, zp[0], zp[1], p0d, p1d, W1[0], W1[1], b1r)
    zp = spmv_k(y, src, dst)
    return _final_call(h, zp[0], zp[1], p0d, p1d, W2[0], W2[1], b2.reshape(1, F))


# speedup vs baseline: 1.0745x; 1.0220x over previous
import functools

import jax
import jax.numpy as jnp
from jax import lax
from jax.experimental import pallas as pl
from jax.experimental.pallas import tpu as pltpu
from jax.experimental.pallas import tpu_sc as plsc

N_NODES = 10000
N_EDGES = 320000
F = 128
NC, NS = 2, 16
CHUNK = 40
EDGES_PER_TILE = N_EDGES // (NC * NS)
NCHUNK = EDGES_PER_TILE // CHUNK
ROWS_A = 624
TAIL = N_NODES - NS * ROWS_A
ZROWS = 208
DEG_W = 16

def _fill_f32(ref, rows, width, value):
    per_row = width // 16

    def body(i, _):
        r = i // per_row
        col = (i % per_row) * 16
        ref[r, pl.ds(col, 16)] = jnp.full((16,), value, jnp.float32)
        return 0

    lax.fori_loop(0, rows * per_row, body, 0)


def _deg_body(dst_hbm, deg_hbm, dst_v, ones_v, zero_v, dsem, deg_sh):
    c = lax.axis_index("c")
    s = lax.axis_index("s")
    _fill_f32(ones_v, CHUNK, DEG_W, 1.0)
    _fill_f32(zero_v, ZROWS, DEG_W, 0.0)
    base = s * ROWS_A
    for k in range(ROWS_A // ZROWS):
        pltpu.sync_copy(zero_v, deg_sh.at[pl.ds(base + k * ZROWS, ZROWS)])

    @pl.when(s == NS - 1)
    def _():
        pltpu.sync_copy(zero_v.at[pl.ds(0, TAIL)],
                        deg_sh.at[pl.ds(NS * ROWS_A, TAIL)])

    pltpu.sync_copy(dst_hbm.at[c, s], dst_v)
    plsc.subcore_barrier()

    GRP = 5
    assert NCHUNK % GRP == 0

    def group(g, _):
        for i in range(GRP):
            pltpu.async_copy(ones_v, deg_sh.at[dst_v.at[g * GRP + i]], dsem,
                             add=True)
        for i in range(GRP):
            pltpu.make_async_copy(ones_v, deg_sh.at[dst_v.at[g * GRP + i]],
                                  dsem).wait()
        return 0

    lax.fori_loop(0, NCHUNK // GRP, group, 0)
    plsc.subcore_barrier()
    pltpu.sync_copy(
        deg_sh.at[pl.ds(base, ROWS_A)],
        deg_hbm.at[c, pl.ds(base, ROWS_A)],
    )

    @pl.when(s == NS - 1)
    def _():
        pltpu.sync_copy(
            deg_sh.at[pl.ds(NS * ROWS_A, TAIL)],
            deg_hbm.at[c, pl.ds(NS * ROWS_A, TAIL)],
        )


NBUF = 6


def _spmv_body(y_hbm, src_hbm, dst_hbm, out_hbm, src_v, dst_v,
               r0, r1, r2, r3, r4, r5, g0, g1, g2, g3, g4, g5,
               t0, t1, t2, t3, t4, t5, z_sh):
    c = lax.axis_index("c")
    s = lax.axis_index("s")
    rows = [r0, r1, r2, r3, r4, r5]
    gsem = [g0, g1, g2, g3, g4, g5]
    ssem = [t0, t1, t2, t3, t4, t5]
    pltpu.async_copy(src_hbm.at[c, s], src_v, gsem[1])
    pltpu.async_copy(dst_hbm.at[c, s], dst_v, gsem[2])
    _fill_f32(r0, CHUNK, F, 0.0)
    base = s * ROWS_A
    nz = ROWS_A // CHUNK
    rem = ROWS_A - nz * CHUNK
    for k in range(nz):
        pltpu.async_copy(r0, z_sh.at[pl.ds(base + k * CHUNK, CHUNK)], gsem[0])
    pltpu.async_copy(r0.at[pl.ds(0, rem)],
                     z_sh.at[pl.ds(base + ROWS_A - rem, rem)], gsem[0])

    @pl.when(s == NS - 1)
    def _():
        pltpu.async_copy(r0.at[pl.ds(0, TAIL)],
                         z_sh.at[pl.ds(NS * ROWS_A, TAIL)], gsem[3])

    for k in range(nz):
        pltpu.make_async_copy(r0, z_sh.at[pl.ds(base + k * CHUNK, CHUNK)],
                              gsem[0]).wait()
    pltpu.make_async_copy(r0.at[pl.ds(0, rem)],
                          z_sh.at[pl.ds(base + ROWS_A - rem, rem)],
                          gsem[0]).wait()

    @pl.when(s == NS - 1)
    def _():
        pltpu.make_async_copy(r0.at[pl.ds(0, TAIL)],
                              z_sh.at[pl.ds(NS * ROWS_A, TAIL)], gsem[3]).wait()

    pltpu.make_async_copy(src_hbm.at[c, s], src_v, gsem[1]).wait()
    pltpu.make_async_copy(dst_hbm.at[c, s], dst_v, gsem[2]).wait()
    plsc.subcore_barrier()

    for b in range(NBUF - 1):
        pltpu.async_copy(y_hbm.at[src_v.at[b]], rows[b], gsem[b])

    def ring(g, _):
        for b in range(NBUF):
            j = g * NBUF + b
            pltpu.make_async_copy(y_hbm.at[src_v.at[j]], rows[b], gsem[b]).wait()
            pltpu.async_copy(rows[b], z_sh.at[dst_v.at[j]], ssem[b], add=True)
            nxt = j + NBUF - 1
            nb = (b + NBUF - 1) % NBUF

            @pl.when((nxt < NCHUNK) & (j >= 1))
            def _():
                pltpu.make_async_copy(rows[nb], z_sh.at[dst_v.at[j - 1]],
                                      ssem[nb]).wait()

            @pl.when(nxt < NCHUNK)
            def _():
                pltpu.async_copy(y_hbm.at[src_v.at[nxt]], rows[nb], gsem[nb])

        return 0

    lax.fori_loop(0, NCHUNK // NBUF, ring, 0)
    for j in range((NCHUNK // NBUF) * NBUF, NCHUNK):
        b = j % NBUF
        pltpu.make_async_copy(y_hbm.at[src_v.at[j]], rows[b], gsem[b]).wait()
        pltpu.async_copy(rows[b], z_sh.at[dst_v.at[j]], ssem[b], add=True)
    last = {}
    for j in range(NCHUNK):
        last[j % NBUF] = j
    for b, j in last.items():
        pltpu.make_async_copy(rows[b], z_sh.at[dst_v.at[j]], ssem[b]).wait()
    plsc.subcore_barrier()
    pltpu.sync_copy(
        z_sh.at[pl.ds(base, ROWS_A)],
        out_hbm.at[c, pl.ds(base, ROWS_A)],
    )

    @pl.when(s == NS - 1)
    def _():
        pltpu.sync_copy(
            z_sh.at[pl.ds(NS * ROWS_A, TAIL)],
            out_hbm.at[c, pl.ds(NS * ROWS_A, TAIL)],
        )


BLK = 1000


def _dinv_of(p0d, p1d):
    d = p0d[:, 0:1] + p1d[:, 0:1]
    return jnp.where(d > 0, lax.rsqrt(jnp.maximum(d, 1.0)), 0.0)


def _prep_body(p0d_ref, p1d_ref, x_ref, y_ref):
    dinv = _dinv_of(p0d_ref[...], p1d_ref[...])
    y_ref[...] = x_ref[...] * dinv


def _layer_body(h_ref, z0_ref, z1_ref, p0d_ref, p1d_ref, w0_ref, w1_ref, b_ref,
                hn_ref, y_ref):
    dinv = _dinv_of(p0d_ref[...], p1d_ref[...])
    ahat = (z0_ref[...] + z1_ref[...]) * dinv
    out = (
        jnp.dot(h_ref[...], w0_ref[...], preferred_element_type=jnp.float32)
        - jnp.dot(ahat, w1_ref[...], preferred_element_type=jnp.float32)
        + b_ref[...]
    )
    hn = jnp.maximum(out, 0.0)
    hn_ref[...] = hn
    y_ref[...] = hn * dinv


def _final_body(h_ref, z0_ref, z1_ref, p0d_ref, p1d_ref, w0_ref, w1_ref, b_ref,
                out_ref):
    dinv = _dinv_of(p0d_ref[...], p1d_ref[...])
    ahat = (z0_ref[...] + z1_ref[...]) * dinv
    out_ref[...] = (
        jnp.dot(h_ref[...], w0_ref[...], preferred_element_type=jnp.float32)
        - jnp.dot(ahat, w1_ref[...], preferred_element_type=jnp.float32)
        + b_ref[...]
    )


_row_blk = lambda w: pl.BlockSpec((BLK, w), lambda i: (i, 0))
_full_w = pl.BlockSpec((F, F), lambda i: (0, 0))
_full_b = pl.BlockSpec((1, F), lambda i: (0, 0))
_GRID = (N_NODES // BLK,)
_f32 = jnp.float32

_prep_call = pl.pallas_call(
    _prep_body,
    grid=_GRID,
    in_specs=[_row_blk(DEG_W), _row_blk(DEG_W), _row_blk(F)],
    out_specs=_row_blk(F),
    out_shape=jax.ShapeDtypeStruct((N_NODES, F), _f32),
)

_layer_call = pl.pallas_call(
    _layer_body,
    grid=_GRID,
    in_specs=[_row_blk(F), _row_blk(F), _row_blk(F), _row_blk(DEG_W),
              _row_blk(DEG_W), _full_w, _full_w, _full_b],
    out_specs=[_row_blk(F), _row_blk(F)],
    out_shape=[jax.ShapeDtypeStruct((N_NODES, F), _f32),
               jax.ShapeDtypeStruct((N_NODES, F), _f32)],
)

_final_call = pl.pallas_call(
    _final_body,
    grid=_GRID,
    in_specs=[_row_blk(F), _row_blk(F), _row_blk(F), _row_blk(DEG_W),
              _row_blk(DEG_W), _full_w, _full_w, _full_b],
    out_specs=_row_blk(F),
    out_shape=jax.ShapeDtypeStruct((N_NODES, F), _f32),
)


@functools.lru_cache(maxsize=1)
def _sc_kernels():
    mesh = plsc.VectorSubcoreMesh(
        core_axis_name="c", subcore_axis_name="s", num_cores=NC, num_subcores=NS
    )
    params = pltpu.CompilerParams(use_tc_tiling_on_sc=False)
    deg_kernel = pl.kernel(
        _deg_body,
        compiler_params=params,
        out_type=jax.ShapeDtypeStruct((NC, N_NODES, DEG_W), jnp.float32),
        mesh=mesh,
        scratch_types=[
            pltpu.VMEM((NCHUNK, CHUNK), jnp.int32),
            pltpu.VMEM((CHUNK, DEG_W), jnp.float32),
            pltpu.VMEM((ZROWS, DEG_W), jnp.float32),
            pltpu.SemaphoreType.DMA,
            pltpu.VMEM_SHARED((N_NODES, DEG_W), jnp.float32),
        ],
    )
    spmv_kernel = pl.kernel(
        _spmv_body,
        compiler_params=params,
        out_type=jax.ShapeDtypeStruct((NC, N_NODES, F), jnp.float32),
        mesh=mesh,
        scratch_types=(
            [pltpu.VMEM((NCHUNK, CHUNK), jnp.int32)] * 2
            + [pltpu.VMEM((CHUNK, F), jnp.float32)] * NBUF
            + [pltpu.SemaphoreType.DMA] * (2 * NBUF)
            + [pltpu.VMEM_SHARED((N_NODES, F), jnp.float32)]
        ),
    )
    return deg_kernel, spmv_kernel


def kernel(x, edge_index, W1, b1, W2, b2):
    deg_k, spmv_k = _sc_kernels()
    src = edge_index[0].astype(jnp.int32).reshape(NC, NS, NCHUNK, CHUNK)
    dst = edge_index[1].astype(jnp.int32).reshape(NC, NS, NCHUNK, CHUNK)

    degp = deg_k(dst)
    p0d, p1d = degp[0], degp[1]

    y = _prep_call(p0d, p1d, x)
    h = x
    b1r = b1.reshape(1, F)
    for _ in range(2):
        zp = spmv_k(y, src, dst)
        h, y = _layer_call(h, zp[0], zp[1], p0d, p1d, W1[0], W1[1], b1r)
    zp = spmv_k(y, src, dst)
    return _final_call(h, zp[0], zp[1], p0d, p1d, W2[0], W2[1], b2.reshape(1, F))


# speedup vs baseline: 1.0829x; 1.0078x over previous
import functools

import jax
import jax.numpy as jnp
from jax import lax
from jax.experimental import pallas as pl
from jax.experimental.pallas import tpu as pltpu
from jax.experimental.pallas import tpu_sc as plsc

N_NODES = 10000
N_EDGES = 320000
F = 128
NC, NS = 2, 16
CHUNK = 40
EDGES_PER_TILE = N_EDGES // (NC * NS)
NCHUNK = EDGES_PER_TILE // CHUNK
ROWS_A = 624
TAIL = N_NODES - NS * ROWS_A
ZROWS = 208
DEG_W = 16

def _fill_f32(ref, rows, width, value):
    per_row = width // 16

    def body(i, _):
        r = i // per_row
        col = (i % per_row) * 16
        ref[r, pl.ds(col, 16)] = jnp.full((16,), value, jnp.float32)
        return 0

    lax.fori_loop(0, rows * per_row, body, 0)


def _deg_body(dst_hbm, deg_hbm, dst_v, ones_v, zero_v, dsem, zsem, deg_sh):
    c = lax.axis_index("c")
    s = lax.axis_index("s")
    pltpu.async_copy(dst_hbm.at[c, s], dst_v, dsem)
    _fill_f32(ones_v, CHUNK, DEG_W, 1.0)
    _fill_f32(zero_v, ZROWS, DEG_W, 0.0)
    base = s * ROWS_A
    for k in range(ROWS_A // ZROWS):
        pltpu.async_copy(zero_v, deg_sh.at[pl.ds(base + k * ZROWS, ZROWS)],
                         zsem)

    @pl.when(s == NS - 1)
    def _():
        pltpu.async_copy(zero_v.at[pl.ds(0, TAIL)],
                         deg_sh.at[pl.ds(NS * ROWS_A, TAIL)], zsem)

    for k in range(ROWS_A // ZROWS):
        pltpu.make_async_copy(zero_v, deg_sh.at[pl.ds(base + k * ZROWS, ZROWS)],
                              zsem).wait()

    @pl.when(s == NS - 1)
    def _():
        pltpu.make_async_copy(zero_v.at[pl.ds(0, TAIL)],
                              deg_sh.at[pl.ds(NS * ROWS_A, TAIL)], zsem).wait()

    pltpu.make_async_copy(dst_hbm.at[c, s], dst_v, dsem).wait()
    plsc.subcore_barrier()

    GRP = 5
    assert NCHUNK % GRP == 0

    def group(g, _):
        for i in range(GRP):
            pltpu.async_copy(ones_v, deg_sh.at[dst_v.at[g * GRP + i]], dsem,
                             add=True)
        for i in range(GRP):
            pltpu.make_async_copy(ones_v, deg_sh.at[dst_v.at[g * GRP + i]],
                                  dsem).wait()
        return 0

    lax.fori_loop(0, NCHUNK // GRP, group, 0)
    plsc.subcore_barrier()
    pltpu.async_copy(deg_sh.at[pl.ds(base, ROWS_A)],
                     deg_hbm.at[c, pl.ds(base, ROWS_A)], zsem)

    @pl.when(s == NS - 1)
    def _():
        pltpu.async_copy(deg_sh.at[pl.ds(NS * ROWS_A, TAIL)],
                         deg_hbm.at[c, pl.ds(NS * ROWS_A, TAIL)], dsem)
        pltpu.make_async_copy(deg_sh.at[pl.ds(NS * ROWS_A, TAIL)],
                              deg_hbm.at[c, pl.ds(NS * ROWS_A, TAIL)],
                              dsem).wait()

    pltpu.make_async_copy(deg_sh.at[pl.ds(base, ROWS_A)],
                          deg_hbm.at[c, pl.ds(base, ROWS_A)], zsem).wait()


NBUF = 6


def _spmv_body(y_hbm, src_hbm, dst_hbm, out_hbm, src_v, dst_v,
               r0, r1, r2, r3, r4, r5, g0, g1, g2, g3, g4, g5,
               t0, t1, t2, t3, t4, t5, z_sh):
    c = lax.axis_index("c")
    s = lax.axis_index("s")
    rows = [r0, r1, r2, r3, r4, r5]
    gsem = [g0, g1, g2, g3, g4, g5]
    ssem = [t0, t1, t2, t3, t4, t5]
    pltpu.async_copy(src_hbm.at[c, s], src_v, gsem[1])
    pltpu.async_copy(dst_hbm.at[c, s], dst_v, gsem[2])
    _fill_f32(r0, CHUNK, F, 0.0)
    base = s * ROWS_A
    nz = ROWS_A // CHUNK
    rem = ROWS_A - nz * CHUNK
    for k in range(nz):
        pltpu.async_copy(r0, z_sh.at[pl.ds(base + k * CHUNK, CHUNK)], gsem[0])
    pltpu.async_copy(r0.at[pl.ds(0, rem)],
                     z_sh.at[pl.ds(base + ROWS_A - rem, rem)], gsem[0])

    @pl.when(s == NS - 1)
    def _():
        pltpu.async_copy(r0.at[pl.ds(0, TAIL)],
                         z_sh.at[pl.ds(NS * ROWS_A, TAIL)], gsem[3])

    for k in range(nz):
        pltpu.make_async_copy(r0, z_sh.at[pl.ds(base + k * CHUNK, CHUNK)],
                              gsem[0]).wait()
    pltpu.make_async_copy(r0.at[pl.ds(0, rem)],
                          z_sh.at[pl.ds(base + ROWS_A - rem, rem)],
                          gsem[0]).wait()

    @pl.when(s == NS - 1)
    def _():
        pltpu.make_async_copy(r0.at[pl.ds(0, TAIL)],
                              z_sh.at[pl.ds(NS * ROWS_A, TAIL)], gsem[3]).wait()

    pltpu.make_async_copy(src_hbm.at[c, s], src_v, gsem[1]).wait()
    pltpu.make_async_copy(dst_hbm.at[c, s], dst_v, gsem[2]).wait()
    plsc.subcore_barrier()

    for b in range(NBUF - 1):
        pltpu.async_copy(y_hbm.at[src_v.at[b]], rows[b], gsem[b])

    def ring(g, _):
        for b in range(NBUF):
            j = g * NBUF + b
            pltpu.make_async_copy(y_hbm.at[src_v.at[j]], rows[b], gsem[b]).wait()
            pltpu.async_copy(rows[b], z_sh.at[dst_v.at[j]], ssem[b], add=True)
            nxt = j + NBUF - 1
            nb = (b + NBUF - 1) % NBUF

            @pl.when((nxt < NCHUNK) & (j >= 1))
            def _():
                pltpu.make_async_copy(rows[nb], z_sh.at[dst_v.at[j - 1]],
                                      ssem[nb]).wait()

            @pl.when(nxt < NCHUNK)
            def _():
                pltpu.async_copy(y_hbm.at[src_v.at[nxt]], rows[nb], gsem[nb])

        return 0

    lax.fori_loop(0, NCHUNK // NBUF, ring, 0)
    for j in range((NCHUNK // NBUF) * NBUF, NCHUNK):
        b = j % NBUF
        pltpu.make_async_copy(y_hbm.at[src_v.at[j]], rows[b], gsem[b]).wait()
        pltpu.async_copy(rows[b], z_sh.at[dst_v.at[j]], ssem[b], add=True)
    last = {}
    for j in range(NCHUNK):
        last[j % NBUF] = j
    for b, j in last.items():
        pltpu.make_async_copy(rows[b], z_sh.at[dst_v.at[j]], ssem[b]).wait()
    plsc.subcore_barrier()
    pltpu.async_copy(z_sh.at[pl.ds(base, ROWS_A)],
                     out_hbm.at[c, pl.ds(base, ROWS_A)], gsem[0])

    @pl.when(s == NS - 1)
    def _():
        pltpu.async_copy(z_sh.at[pl.ds(NS * ROWS_A, TAIL)],
                         out_hbm.at[c, pl.ds(NS * ROWS_A, TAIL)], gsem[1])
        pltpu.make_async_copy(z_sh.at[pl.ds(NS * ROWS_A, TAIL)],
                              out_hbm.at[c, pl.ds(NS * ROWS_A, TAIL)],
                              gsem[1]).wait()

    pltpu.make_async_copy(z_sh.at[pl.ds(base, ROWS_A)],
                          out_hbm.at[c, pl.ds(base, ROWS_A)], gsem[0]).wait()


BLK = 1000


def _dinv_of(p0d, p1d):
    d = p0d[:, 0:1] + p1d[:, 0:1]
    return jnp.where(d > 0, lax.rsqrt(jnp.maximum(d, 1.0)), 0.0)


def _prep_body(p0d_ref, p1d_ref, x_ref, y_ref):
    dinv = _dinv_of(p0d_ref[...], p1d_ref[...])
    y_ref[...] = x_ref[...] * dinv


def _layer_body(h_ref, z0_ref, z1_ref, p0d_ref, p1d_ref, w0_ref, w1_ref, b_ref,
                hn_ref, y_ref):
    dinv = _dinv_of(p0d_ref[...], p1d_ref[...])
    ahat = (z0_ref[...] + z1_ref[...]) * dinv
    out = (
        jnp.dot(h_ref[...], w0_ref[...], preferred_element_type=jnp.float32)
        - jnp.dot(ahat, w1_ref[...], preferred_element_type=jnp.float32)
        + b_ref[...]
    )
    hn = jnp.maximum(out, 0.0)
    hn_ref[...] = hn
    y_ref[...] = hn * dinv


def _final_body(h_ref, z0_ref, z1_ref, p0d_ref, p1d_ref, w0_ref, w1_ref, b_ref,
                out_ref):
    dinv = _dinv_of(p0d_ref[...], p1d_ref[...])
    ahat = (z0_ref[...] + z1_ref[...]) * dinv
    out_ref[...] = (
        jnp.dot(h_ref[...], w0_ref[...], preferred_element_type=jnp.float32)
        - jnp.dot(ahat, w1_ref[...], preferred_element_type=jnp.float32)
        + b_ref[...]
    )


_row_blk = lambda w: pl.BlockSpec((BLK, w), lambda i: (i, 0))
_full_w = pl.BlockSpec((F, F), lambda i: (0, 0))
_full_b = pl.BlockSpec((1, F), lambda i: (0, 0))
_GRID = (N_NODES // BLK,)
_f32 = jnp.float32

_prep_call = pl.pallas_call(
    _prep_body,
    grid=_GRID,
    in_specs=[_row_blk(DEG_W), _row_blk(DEG_W), _row_blk(F)],
    out_specs=_row_blk(F),
    out_shape=jax.ShapeDtypeStruct((N_NODES, F), _f32),
)

_layer_call = pl.pallas_call(
    _layer_body,
    grid=_GRID,
    in_specs=[_row_blk(F), _row_blk(F), _row_blk(F), _row_blk(DEG_W),
              _row_blk(DEG_W), _full_w, _full_w, _full_b],
    out_specs=[_row_blk(F), _row_blk(F)],
    out_shape=[jax.ShapeDtypeStruct((N_NODES, F), _f32),
               jax.ShapeDtypeStruct((N_NODES, F), _f32)],
)

_final_call = pl.pallas_call(
    _final_body,
    grid=_GRID,
    in_specs=[_row_blk(F), _row_blk(F), _row_blk(F), _row_blk(DEG_W),
              _row_blk(DEG_W), _full_w, _full_w, _full_b],
    out_specs=_row_blk(F),
    out_shape=jax.ShapeDtypeStruct((N_NODES, F), _f32),
)


@functools.lru_cache(maxsize=1)
def _sc_kernels():
    mesh = plsc.VectorSubcoreMesh(
        core_axis_name="c", subcore_axis_name="s", num_cores=NC, num_subcores=NS
    )
    params = pltpu.CompilerParams(use_tc_tiling_on_sc=False)
    deg_kernel = pl.kernel(
        _deg_body,
        compiler_params=params,
        out_type=jax.ShapeDtypeStruct((NC, N_NODES, DEG_W), jnp.float32),
        mesh=mesh,
        scratch_types=[
            pltpu.VMEM((NCHUNK, CHUNK), jnp.int32),
            pltpu.VMEM((CHUNK, DEG_W), jnp.float32),
            pltpu.VMEM((ZROWS, DEG_W), jnp.float32),
            pltpu.SemaphoreType.DMA,
            pltpu.SemaphoreType.DMA,
            pltpu.VMEM_SHARED((N_NODES, DEG_W), jnp.float32),
        ],
    )
    spmv_kernel = pl.kernel(
        _spmv_body,
        compiler_params=params,
        out_type=jax.ShapeDtypeStruct((NC, N_NODES, F), jnp.float32),
        mesh=mesh,
        scratch_types=(
            [pltpu.VMEM((NCHUNK, CHUNK), jnp.int32)] * 2
            + [pltpu.VMEM((CHUNK, F), jnp.float32)] * NBUF
            + [pltpu.SemaphoreType.DMA] * (2 * NBUF)
            + [pltpu.VMEM_SHARED((N_NODES, F), jnp.float32)]
        ),
    )
    return deg_kernel, spmv_kernel


def kernel(x, edge_index, W1, b1, W2, b2):
    deg_k, spmv_k = _sc_kernels()
    src = edge_index[0].astype(jnp.int32).reshape(NC, NS, NCHUNK, CHUNK)
    dst = edge_index[1].astype(jnp.int32).reshape(NC, NS, NCHUNK, CHUNK)

    degp = deg_k(dst)
    p0d, p1d = degp[0], degp[1]

    y = _prep_call(p0d, p1d, x)
    h = x
    b1r = b1.reshape(1, F)
    for _ in range(2):
        zp = spmv_k(y, src, dst)
        h, y = _layer_call(h, zp[0], zp[1], p0d, p1d, W1[0], W1[1], b1r)
    zp = spmv_k(y, src, dst)
    return _final_call(h, zp[0], zp[1], p0d, p1d, W2[0], W2[1], b2.reshape(1, F))


# speedup vs baseline: 1.1021x; 1.0177x over previous
import functools

import jax
import jax.numpy as jnp
from jax import lax
from jax.experimental import pallas as pl
from jax.experimental.pallas import tpu as pltpu
from jax.experimental.pallas import tpu_sc as plsc

N_NODES = 10000
N_EDGES = 320000
F = 128
NC, NS = 2, 16
CHUNK = 40
EDGES_PER_TILE = N_EDGES // (NC * NS)
NCHUNK = EDGES_PER_TILE // CHUNK
ROWS_A = 624
TAIL = N_NODES - NS * ROWS_A
ZROWS = 208
DEG_W = 16

def _fill_f32(ref, rows, width, value):
    per_row = width // 16

    def body(i, _):
        r = i // per_row
        col = (i % per_row) * 16
        ref[r, pl.ds(col, 16)] = jnp.full((16,), value, jnp.float32)
        return 0

    lax.fori_loop(0, rows * per_row, body, 0)


def _deg_body(dst_hbm, deg_hbm, dst_v, ones_v, zero_v, dsem, zsem, deg_sh):
    c = lax.axis_index("c")
    s = lax.axis_index("s")
    pltpu.async_copy(dst_hbm.at[c, s], dst_v, dsem)
    _fill_f32(ones_v, CHUNK, DEG_W, 1.0)
    _fill_f32(zero_v, ZROWS, DEG_W, 0.0)
    base = s * ROWS_A
    for k in range(ROWS_A // ZROWS):
        pltpu.async_copy(zero_v, deg_sh.at[pl.ds(base + k * ZROWS, ZROWS)],
                         zsem)

    @pl.when(s == NS - 1)
    def _():
        pltpu.async_copy(zero_v.at[pl.ds(0, TAIL)],
                         deg_sh.at[pl.ds(NS * ROWS_A, TAIL)], zsem)

    for k in range(ROWS_A // ZROWS):
        pltpu.make_async_copy(zero_v, deg_sh.at[pl.ds(base + k * ZROWS, ZROWS)],
                              zsem).wait()

    @pl.when(s == NS - 1)
    def _():
        pltpu.make_async_copy(zero_v.at[pl.ds(0, TAIL)],
                              deg_sh.at[pl.ds(NS * ROWS_A, TAIL)], zsem).wait()

    pltpu.make_async_copy(dst_hbm.at[c, s], dst_v, dsem).wait()
    plsc.subcore_barrier()

    GRP = 5
    assert NCHUNK % GRP == 0

    def group(g, _):
        for i in range(GRP):
            pltpu.async_copy(ones_v, deg_sh.at[dst_v.at[g * GRP + i]], dsem,
                             add=True)
        for i in range(GRP):
            pltpu.make_async_copy(ones_v, deg_sh.at[dst_v.at[g * GRP + i]],
                                  dsem).wait()
        return 0

    lax.fori_loop(0, NCHUNK // GRP, group, 0)
    plsc.subcore_barrier()
    pltpu.async_copy(deg_sh.at[pl.ds(base, ROWS_A)],
                     deg_hbm.at[c, pl.ds(base, ROWS_A)], zsem)

    @pl.when(s == NS - 1)
    def _():
        pltpu.async_copy(deg_sh.at[pl.ds(NS * ROWS_A, TAIL)],
                         deg_hbm.at[c, pl.ds(NS * ROWS_A, TAIL)], dsem)
        pltpu.make_async_copy(deg_sh.at[pl.ds(NS * ROWS_A, TAIL)],
                              deg_hbm.at[c, pl.ds(NS * ROWS_A, TAIL)],
                              dsem).wait()

    pltpu.make_async_copy(deg_sh.at[pl.ds(base, ROWS_A)],
                          deg_hbm.at[c, pl.ds(base, ROWS_A)], zsem).wait()


NBUF = 6


def _spmv_body(y_hbm, src_hbm, dst_hbm, out_hbm, src_v, dst_v,
               r0, r1, r2, r3, r4, r5, g0, g1, g2, g3, g4, g5,
               t0, t1, t2, t3, t4, t5, z_sh):
    c = lax.axis_index("c")
    s = lax.axis_index("s")
    rows = [r0, r1, r2, r3, r4, r5]
    gsem = [g0, g1, g2, g3, g4, g5]
    ssem = [t0, t1, t2, t3, t4, t5]
    pltpu.async_copy(src_hbm.at[c, s], src_v, gsem[1])
    pltpu.async_copy(dst_hbm.at[c, s], dst_v, gsem[2])
    _fill_f32(r0, CHUNK, F, 0.0)
    base = s * ROWS_A
    nz = ROWS_A // CHUNK
    rem = ROWS_A - nz * CHUNK
    for k in range(nz):
        pltpu.async_copy(r0, z_sh.at[pl.ds(base + k * CHUNK, CHUNK)], gsem[0])
    pltpu.async_copy(r0.at[pl.ds(0, rem)],
                     z_sh.at[pl.ds(base + ROWS_A - rem, rem)], gsem[0])

    @pl.when(s == NS - 1)
    def _():
        pltpu.async_copy(r0.at[pl.ds(0, TAIL)],
                         z_sh.at[pl.ds(NS * ROWS_A, TAIL)], gsem[3])

    for k in range(nz):
        pltpu.make_async_copy(r0, z_sh.at[pl.ds(base + k * CHUNK, CHUNK)],
                              gsem[0]).wait()
    pltpu.make_async_copy(r0.at[pl.ds(0, rem)],
                          z_sh.at[pl.ds(base + ROWS_A - rem, rem)],
                          gsem[0]).wait()

    @pl.when(s == NS - 1)
    def _():
        pltpu.make_async_copy(r0.at[pl.ds(0, TAIL)],
                              z_sh.at[pl.ds(NS * ROWS_A, TAIL)], gsem[3]).wait()

    pltpu.make_async_copy(src_hbm.at[c, s], src_v, gsem[1]).wait()
    pltpu.make_async_copy(dst_hbm.at[c, s], dst_v, gsem[2]).wait()
    plsc.subcore_barrier()

    for b in range(NBUF - 1):
        pltpu.async_copy(y_hbm.at[src_v.at[b]], rows[b], gsem[b])

    def ring(g, _):
        for b in range(NBUF):
            j = g * NBUF + b
            pltpu.make_async_copy(y_hbm.at[src_v.at[j]], rows[b], gsem[b]).wait()
            pltpu.async_copy(rows[b], z_sh.at[dst_v.at[j]], ssem[b], add=True)
            nxt = j + NBUF - 1
            nb = (b + NBUF - 1) % NBUF

            @pl.when((nxt < NCHUNK) & (j >= 1))
            def _():
                pltpu.make_async_copy(rows[nb], z_sh.at[dst_v.at[j - 1]],
                                      ssem[nb]).wait()

            @pl.when(nxt < NCHUNK)
            def _():
                pltpu.async_copy(y_hbm.at[src_v.at[nxt]], rows[nb], gsem[nb])

        return 0

    lax.fori_loop(0, NCHUNK // NBUF, ring, 0)
    for j in range((NCHUNK // NBUF) * NBUF, NCHUNK):
        b = j % NBUF
        pltpu.make_async_copy(y_hbm.at[src_v.at[j]], rows[b], gsem[b]).wait()
        pltpu.async_copy(rows[b], z_sh.at[dst_v.at[j]], ssem[b], add=True)
    last = {}
    for j in range(NCHUNK):
        last[j % NBUF] = j
    for b, j in last.items():
        pltpu.make_async_copy(rows[b], z_sh.at[dst_v.at[j]], ssem[b]).wait()
    plsc.subcore_barrier()
    pltpu.async_copy(z_sh.at[pl.ds(base, ROWS_A)],
                     out_hbm.at[c, pl.ds(base, ROWS_A)], gsem[0])

    @pl.when(s == NS - 1)
    def _():
        pltpu.async_copy(z_sh.at[pl.ds(NS * ROWS_A, TAIL)],
                         out_hbm.at[c, pl.ds(NS * ROWS_A, TAIL)], gsem[1])
        pltpu.make_async_copy(z_sh.at[pl.ds(NS * ROWS_A, TAIL)],
                              out_hbm.at[c, pl.ds(NS * ROWS_A, TAIL)],
                              gsem[1]).wait()

    pltpu.make_async_copy(z_sh.at[pl.ds(base, ROWS_A)],
                          out_hbm.at[c, pl.ds(base, ROWS_A)], gsem[0]).wait()


BLK = 2000


def _dinv_of(p0d, p1d):
    d = p0d[:, 0:1] + p1d[:, 0:1]
    return jnp.where(d > 0, lax.rsqrt(jnp.maximum(d, 1.0)), 0.0)


def _prep_body(p0d_ref, p1d_ref, x_ref, y_ref):
    dinv = _dinv_of(p0d_ref[...], p1d_ref[...])
    y_ref[...] = x_ref[...] * dinv


def _layer_body(h_ref, z0_ref, z1_ref, p0d_ref, p1d_ref, w0_ref, w1_ref, b_ref,
                hn_ref, y_ref):
    dinv = _dinv_of(p0d_ref[...], p1d_ref[...])
    ahat = (z0_ref[...] + z1_ref[...]) * dinv
    out = (
        jnp.dot(h_ref[...], w0_ref[...], preferred_element_type=jnp.float32)
        - jnp.dot(ahat, w1_ref[...], preferred_element_type=jnp.float32)
        + b_ref[...]
    )
    hn = jnp.maximum(out, 0.0)
    hn_ref[...] = hn
    y_ref[...] = hn * dinv


def _final_body(h_ref, z0_ref, z1_ref, p0d_ref, p1d_ref, w0_ref, w1_ref, b_ref,
                out_ref):
    dinv = _dinv_of(p0d_ref[...], p1d_ref[...])
    ahat = (z0_ref[...] + z1_ref[...]) * dinv
    out_ref[...] = (
        jnp.dot(h_ref[...], w0_ref[...], preferred_element_type=jnp.float32)
        - jnp.dot(ahat, w1_ref[...], preferred_element_type=jnp.float32)
        + b_ref[...]
    )


_row_blk = lambda w: pl.BlockSpec((BLK, w), lambda i: (i, 0))
_full_w = pl.BlockSpec((F, F), lambda i: (0, 0))
_full_b = pl.BlockSpec((1, F), lambda i: (0, 0))
_GRID = (N_NODES // BLK,)
_f32 = jnp.float32

_prep_call = pl.pallas_call(
    _prep_body,
    grid=_GRID,
    in_specs=[_row_blk(DEG_W), _row_blk(DEG_W), _row_blk(F)],
    out_specs=_row_blk(F),
    out_shape=jax.ShapeDtypeStruct((N_NODES, F), _f32),
)

_layer_call = pl.pallas_call(
    _layer_body,
    grid=_GRID,
    in_specs=[_row_blk(F), _row_blk(F), _row_blk(F), _row_blk(DEG_W),
              _row_blk(DEG_W), _full_w, _full_w, _full_b],
    out_specs=[_row_blk(F), _row_blk(F)],
    out_shape=[jax.ShapeDtypeStruct((N_NODES, F), _f32),
               jax.ShapeDtypeStruct((N_NODES, F), _f32)],
)

_final_call = pl.pallas_call(
    _final_body,
    grid=_GRID,
    in_specs=[_row_blk(F), _row_blk(F), _row_blk(F), _row_blk(DEG_W),
              _row_blk(DEG_W), _full_w, _full_w, _full_b],
    out_specs=_row_blk(F),
    out_shape=jax.ShapeDtypeStruct((N_NODES, F), _f32),
)


@functools.lru_cache(maxsize=1)
def _sc_kernels():
    mesh = plsc.VectorSubcoreMesh(
        core_axis_name="c", subcore_axis_name="s", num_cores=NC, num_subcores=NS
    )
    params = pltpu.CompilerParams(use_tc_tiling_on_sc=False)
    deg_kernel = pl.kernel(
        _deg_body,
        compiler_params=params,
        out_type=jax.ShapeDtypeStruct((NC, N_NODES, DEG_W), jnp.float32),
        mesh=mesh,
        scratch_types=[
            pltpu.VMEM((NCHUNK, CHUNK), jnp.int32),
            pltpu.VMEM((CHUNK, DEG_W), jnp.float32),
            pltpu.VMEM((ZROWS, DEG_W), jnp.float32),
            pltpu.SemaphoreType.DMA,
            pltpu.SemaphoreType.DMA,
            pltpu.VMEM_SHARED((N_NODES, DEG_W), jnp.float32),
        ],
    )
    spmv_kernel = pl.kernel(
        _spmv_body,
        compiler_params=params,
        out_type=jax.ShapeDtypeStruct((NC, N_NODES, F), jnp.float32),
        mesh=mesh,
        scratch_types=(
            [pltpu.VMEM((NCHUNK, CHUNK), jnp.int32)] * 2
            + [pltpu.VMEM((CHUNK, F), jnp.float32)] * NBUF
            + [pltpu.SemaphoreType.DMA] * (2 * NBUF)
            + [pltpu.VMEM_SHARED((N_NODES, F), jnp.float32)]
        ),
    )
    return deg_kernel, spmv_kernel


def kernel(x, edge_index, W1, b1, W2, b2):
    deg_k, spmv_k = _sc_kernels()
    src = edge_index[0].astype(jnp.int32).reshape(NC, NS, NCHUNK, CHUNK)
    dst = edge_index[1].astype(jnp.int32).reshape(NC, NS, NCHUNK, CHUNK)

    degp = deg_k(dst)
    p0d, p1d = degp[0], degp[1]

    y = _prep_call(p0d, p1d, x)
    h = x
    b1r = b1.reshape(1, F)
    for _ in range(2):
        zp = spmv_k(y, src, dst)
        h, y = _layer_call(h, zp[0], zp[1], p0d, p1d, W1[0], W1[1], b1r)
    zp = spmv_k(y, src, dst)
    return _final_call(h, zp[0], zp[1], p0d, p1d, W2[0], W2[1], b2.reshape(1, F))


# speedup vs baseline: 1.1193x; 1.0156x over previous
import functools

import jax
import jax.numpy as jnp
from jax import lax
from jax.experimental import pallas as pl
from jax.experimental.pallas import tpu as pltpu
from jax.experimental.pallas import tpu_sc as plsc

N_NODES = 10000
N_EDGES = 320000
F = 128
NC, NS = 2, 16
CHUNK = 40
EDGES_PER_TILE = N_EDGES // (NC * NS)
NCHUNK = EDGES_PER_TILE // CHUNK
ROWS_A = 624
TAIL = N_NODES - NS * ROWS_A
ZROWS = 208
DEG_W = 16

def _fill_f32(ref, rows, width, value):
    per_row = width // 16

    def body(i, _):
        r = i // per_row
        col = (i % per_row) * 16
        ref[r, pl.ds(col, 16)] = jnp.full((16,), value, jnp.float32)
        return 0

    lax.fori_loop(0, rows * per_row, body, 0)


def _deg_body(dst_hbm, deg_hbm, dst_v, ones_v, zero_v, dsem, zsem, deg_sh):
    c = lax.axis_index("c")
    s = lax.axis_index("s")
    pltpu.async_copy(dst_hbm.at[c, s], dst_v, dsem)
    _fill_f32(ones_v, CHUNK, DEG_W, 1.0)
    _fill_f32(zero_v, ZROWS, DEG_W, 0.0)
    base = s * ROWS_A
    for k in range(ROWS_A // ZROWS):
        pltpu.async_copy(zero_v, deg_sh.at[pl.ds(base + k * ZROWS, ZROWS)],
                         zsem)

    @pl.when(s == NS - 1)
    def _():
        pltpu.async_copy(zero_v.at[pl.ds(0, TAIL)],
                         deg_sh.at[pl.ds(NS * ROWS_A, TAIL)], zsem)

    for k in range(ROWS_A // ZROWS):
        pltpu.make_async_copy(zero_v, deg_sh.at[pl.ds(base + k * ZROWS, ZROWS)],
                              zsem).wait()

    @pl.when(s == NS - 1)
    def _():
        pltpu.make_async_copy(zero_v.at[pl.ds(0, TAIL)],
                              deg_sh.at[pl.ds(NS * ROWS_A, TAIL)], zsem).wait()

    pltpu.make_async_copy(dst_hbm.at[c, s], dst_v, dsem).wait()
    plsc.subcore_barrier()

    GRP = 5
    assert NCHUNK % GRP == 0

    def group(g, _):
        for i in range(GRP):
            pltpu.async_copy(ones_v, deg_sh.at[dst_v.at[g * GRP + i]], dsem,
                             add=True)
        for i in range(GRP):
            pltpu.make_async_copy(ones_v, deg_sh.at[dst_v.at[g * GRP + i]],
                                  dsem).wait()
        return 0

    lax.fori_loop(0, NCHUNK // GRP, group, 0)
    plsc.subcore_barrier()
    pltpu.async_copy(deg_sh.at[pl.ds(base, ROWS_A)],
                     deg_hbm.at[c, pl.ds(base, ROWS_A)], zsem)

    @pl.when(s == NS - 1)
    def _():
        pltpu.async_copy(deg_sh.at[pl.ds(NS * ROWS_A, TAIL)],
                         deg_hbm.at[c, pl.ds(NS * ROWS_A, TAIL)], dsem)
        pltpu.make_async_copy(deg_sh.at[pl.ds(NS * ROWS_A, TAIL)],
                              deg_hbm.at[c, pl.ds(NS * ROWS_A, TAIL)],
                              dsem).wait()

    pltpu.make_async_copy(deg_sh.at[pl.ds(base, ROWS_A)],
                          deg_hbm.at[c, pl.ds(base, ROWS_A)], zsem).wait()


NBUF = 6


def _spmv_body(y_hbm, src_hbm, dst_hbm, out_hbm, src_v, dst_v,
               r0, r1, r2, r3, r4, r5, g0, g1, g2, g3, g4, g5,
               t0, t1, t2, t3, t4, t5, z_sh):
    c = lax.axis_index("c")
    s = lax.axis_index("s")
    rows = [r0, r1, r2, r3, r4, r5]
    gsem = [g0, g1, g2, g3, g4, g5]
    ssem = [t0, t1, t2, t3, t4, t5]
    pltpu.async_copy(src_hbm.at[c, s], src_v, gsem[1])
    pltpu.async_copy(dst_hbm.at[c, s], dst_v, gsem[2])
    _fill_f32(r0, CHUNK, F, 0.0)
    base = s * ROWS_A
    nz = ROWS_A // CHUNK
    rem = ROWS_A - nz * CHUNK
    for k in range(nz):
        pltpu.async_copy(r0, z_sh.at[pl.ds(base + k * CHUNK, CHUNK)], gsem[0])
    pltpu.async_copy(r0.at[pl.ds(0, rem)],
                     z_sh.at[pl.ds(base + ROWS_A - rem, rem)], gsem[0])

    @pl.when(s == NS - 1)
    def _():
        pltpu.async_copy(r0.at[pl.ds(0, TAIL)],
                         z_sh.at[pl.ds(NS * ROWS_A, TAIL)], gsem[3])

    for k in range(nz):
        pltpu.make_async_copy(r0, z_sh.at[pl.ds(base + k * CHUNK, CHUNK)],
                              gsem[0]).wait()
    pltpu.make_async_copy(r0.at[pl.ds(0, rem)],
                          z_sh.at[pl.ds(base + ROWS_A - rem, rem)],
                          gsem[0]).wait()

    @pl.when(s == NS - 1)
    def _():
        pltpu.make_async_copy(r0.at[pl.ds(0, TAIL)],
                              z_sh.at[pl.ds(NS * ROWS_A, TAIL)], gsem[3]).wait()

    pltpu.make_async_copy(src_hbm.at[c, s], src_v, gsem[1]).wait()
    pltpu.make_async_copy(dst_hbm.at[c, s], dst_v, gsem[2]).wait()
    plsc.subcore_barrier()

    for b in range(NBUF - 1):
        pltpu.async_copy(y_hbm.at[src_v.at[b]], rows[b], gsem[b])

    def ring(g, _):
        for b in range(NBUF):
            j = g * NBUF + b
            pltpu.make_async_copy(y_hbm.at[src_v.at[j]], rows[b], gsem[b]).wait()
            pltpu.async_copy(rows[b], z_sh.at[dst_v.at[j]], ssem[b], add=True)
            nxt = j + NBUF - 1
            nb = (b + NBUF - 1) % NBUF

            @pl.when((nxt < NCHUNK) & (j >= 1))
            def _():
                pltpu.make_async_copy(rows[nb], z_sh.at[dst_v.at[j - 1]],
                                      ssem[nb]).wait()

            @pl.when(nxt < NCHUNK)
            def _():
                pltpu.async_copy(y_hbm.at[src_v.at[nxt]], rows[nb], gsem[nb])

        return 0

    lax.fori_loop(0, NCHUNK // NBUF, ring, 0)
    for j in range((NCHUNK // NBUF) * NBUF, NCHUNK):
        b = j % NBUF
        pltpu.make_async_copy(y_hbm.at[src_v.at[j]], rows[b], gsem[b]).wait()
        pltpu.async_copy(rows[b], z_sh.at[dst_v.at[j]], ssem[b], add=True)
    last = {}
    for j in range(NCHUNK):
        last[j % NBUF] = j
    for b, j in last.items():
        pltpu.make_async_copy(rows[b], z_sh.at[dst_v.at[j]], ssem[b]).wait()
    plsc.subcore_barrier()
    pltpu.async_copy(z_sh.at[pl.ds(base, ROWS_A)],
                     out_hbm.at[c, pl.ds(base, ROWS_A)], gsem[0])

    @pl.when(s == NS - 1)
    def _():
        pltpu.async_copy(z_sh.at[pl.ds(NS * ROWS_A, TAIL)],
                         out_hbm.at[c, pl.ds(NS * ROWS_A, TAIL)], gsem[1])
        pltpu.make_async_copy(z_sh.at[pl.ds(NS * ROWS_A, TAIL)],
                              out_hbm.at[c, pl.ds(NS * ROWS_A, TAIL)],
                              gsem[1]).wait()

    pltpu.make_async_copy(z_sh.at[pl.ds(base, ROWS_A)],
                          out_hbm.at[c, pl.ds(base, ROWS_A)], gsem[0]).wait()


BLK = 5000


def _dinv_of(p0d, p1d):
    d = p0d[:, 0:1] + p1d[:, 0:1]
    return jnp.where(d > 0, lax.rsqrt(jnp.maximum(d, 1.0)), 0.0)


def _prep_body(p0d_ref, p1d_ref, x_ref, y_ref):
    dinv = _dinv_of(p0d_ref[...], p1d_ref[...])
    y_ref[...] = x_ref[...] * dinv


def _layer_body(h_ref, z0_ref, z1_ref, p0d_ref, p1d_ref, w0_ref, w1_ref, b_ref,
                hn_ref, y_ref):
    dinv = _dinv_of(p0d_ref[...], p1d_ref[...])
    ahat = (z0_ref[...] + z1_ref[...]) * dinv
    out = (
        jnp.dot(h_ref[...], w0_ref[...], preferred_element_type=jnp.float32)
        - jnp.dot(ahat, w1_ref[...], preferred_element_type=jnp.float32)
        + b_ref[...]
    )
    hn = jnp.maximum(out, 0.0)
    hn_ref[...] = hn
    y_ref[...] = hn * dinv


def _final_body(h_ref, z0_ref, z1_ref, p0d_ref, p1d_ref, w0_ref, w1_ref, b_ref,
                out_ref):
    dinv = _dinv_of(p0d_ref[...], p1d_ref[...])
    ahat = (z0_ref[...] + z1_ref[...]) * dinv
    out_ref[...] = (
        jnp.dot(h_ref[...], w0_ref[...], preferred_element_type=jnp.float32)
        - jnp.dot(ahat, w1_ref[...], preferred_element_type=jnp.float32)
        + b_ref[...]
    )


_row_blk = lambda w: pl.BlockSpec((BLK, w), lambda i: (i, 0))
_full_w = pl.BlockSpec((F, F), lambda i: (0, 0))
_full_b = pl.BlockSpec((1, F), lambda i: (0, 0))
_GRID = (N_NODES // BLK,)
_f32 = jnp.float32

_prep_call = pl.pallas_call(
    _prep_body,
    grid=_GRID,
    in_specs=[_row_blk(DEG_W), _row_blk(DEG_W), _row_blk(F)],
    out_specs=_row_blk(F),
    out_shape=jax.ShapeDtypeStruct((N_NODES, F), _f32),
)

_layer_call = pl.pallas_call(
    _layer_body,
    grid=_GRID,
    in_specs=[_row_blk(F), _row_blk(F), _row_blk(F), _row_blk(DEG_W),
              _row_blk(DEG_W), _full_w, _full_w, _full_b],
    out_specs=[_row_blk(F), _row_blk(F)],
    out_shape=[jax.ShapeDtypeStruct((N_NODES, F), _f32),
               jax.ShapeDtypeStruct((N_NODES, F), _f32)],
)

_final_call = pl.pallas_call(
    _final_body,
    grid=_GRID,
    in_specs=[_row_blk(F), _row_blk(F), _row_blk(F), _row_blk(DEG_W),
              _row_blk(DEG_W), _full_w, _full_w, _full_b],
    out_specs=_row_blk(F),
    out_shape=jax.ShapeDtypeStruct((N_NODES, F), _f32),
)


@functools.lru_cache(maxsize=1)
def _sc_kernels():
    mesh = plsc.VectorSubcoreMesh(
        core_axis_name="c", subcore_axis_name="s", num_cores=NC, num_subcores=NS
    )
    params = pltpu.CompilerParams(use_tc_tiling_on_sc=False)
    deg_kernel = pl.kernel(
        _deg_body,
        compiler_params=params,
        out_type=jax.ShapeDtypeStruct((NC, N_NODES, DEG_W), jnp.float32),
        mesh=mesh,
        scratch_types=[
            pltpu.VMEM((NCHUNK, CHUNK), jnp.int32),
            pltpu.VMEM((CHUNK, DEG_W), jnp.float32),
            pltpu.VMEM((ZROWS, DEG_W), jnp.float32),
            pltpu.SemaphoreType.DMA,
            pltpu.SemaphoreType.DMA,
            pltpu.VMEM_SHARED((N_NODES, DEG_W), jnp.float32),
        ],
    )
    spmv_kernel = pl.kernel(
        _spmv_body,
        compiler_params=params,
        out_type=jax.ShapeDtypeStruct((NC, N_NODES, F), jnp.float32),
        mesh=mesh,
        scratch_types=(
            [pltpu.VMEM((NCHUNK, CHUNK), jnp.int32)] * 2
            + [pltpu.VMEM((CHUNK, F), jnp.float32)] * NBUF
            + [pltpu.SemaphoreType.DMA] * (2 * NBUF)
            + [pltpu.VMEM_SHARED((N_NODES, F), jnp.float32)]
        ),
    )
    return deg_kernel, spmv_kernel


def kernel(x, edge_index, W1, b1, W2, b2):
    deg_k, spmv_k = _sc_kernels()
    src = edge_index[0].astype(jnp.int32).reshape(NC, NS, NCHUNK, CHUNK)
    dst = edge_index[1].astype(jnp.int32).reshape(NC, NS, NCHUNK, CHUNK)

    degp = deg_k(dst)
    p0d, p1d = degp[0], degp[1]

    y = _prep_call(p0d, p1d, x)
    h = x
    b1r = b1.reshape(1, F)
    for _ in range(2):
        zp = spmv_k(y, src, dst)
        h, y = _layer_call(h, zp[0], zp[1], p0d, p1d, W1[0], W1[1], b1r)
    zp = spmv_k(y, src, dst)
    return _final_call(h, zp[0], zp[1], p0d, p1d, W2[0], W2[1], b2.reshape(1, F))
